# baseline (speedup 1.0000x reference)
import functools

import jax
import jax.numpy as jnp
from jax import lax
from jax.experimental import pallas as pl
from jax.experimental.pallas import tpu as pltpu
from jax.experimental.pallas import tpu_sc as plsc

_N = 10000
_E = 320000
_D = 128
_DE = 16
_L = 3
_BN_EPS = 1e-5

_CH = 128
_K = 2
_CB = _CH * _K
_NCHUNK = _E // _CB
_TILES = 16
_HALF = _N // 2
_Q = (2504, 2496)
_AGR = 2512
_TRASH = 2504
_ZPT = _AGR // _TILES
_RPT = 152

_BE = 8000


def _edge_enc(edge_attr, We, be3):
    def body(ea_ref, w_ref, b_ref, o_ref):
        o_ref[...] = (
            jnp.dot(ea_ref[...], w_ref[0], preferred_element_type=jnp.float32)
            + b_ref[0]
        )

    nblk = _E // _BE
    return pl.pallas_call(
        body,
        grid=(_L, nblk),
        in_specs=[
            pl.BlockSpec((_BE, _DE), lambda l, i: (i, 0)),
            pl.BlockSpec((1, _DE, _D), lambda l, i: (l, 0, 0)),
            pl.BlockSpec((1, 1, _D), lambda l, i: (l, 0, 0)),
        ],
        out_specs=pl.BlockSpec((_BE, _D), lambda l, i: (l * nblk + i, 0)),
        out_shape=jax.ShapeDtypeStruct((_L * _E, _D), jnp.float32),
    )(edge_attr, We, be3)


def _update(x, aggr, prev, Wl, bl, gl, betal):
    def body(x_ref, a_ref, p_ref, w_ref, b_ref, g_ref, bb_ref,
             hp_ref, xf_ref):
        h = (
            jnp.dot(x_ref[...] + a_ref[...], w_ref[...],
                    preferred_element_type=jnp.float32)
            + b_ref[...]
        )
        m = jnp.mean(h, axis=0, keepdims=True)
        v = jnp.mean((h - m) ** 2, axis=0, keepdims=True)
        hn = (h - m) * lax.rsqrt(v + _BN_EPS) * g_ref[...] + bb_ref[...]
        hp = hn + p_ref[...]
        hp_ref[...] = hp
        xf_ref[...] = jnp.maximum(hp, 0.0)

    return pl.pallas_call(
        body,
        out_shape=(
            jax.ShapeDtypeStruct((_N, _D), jnp.float32),
            jax.ShapeDtypeStruct((_N, _D), jnp.float32),
        ),
    )(x, aggr, prev, Wl, bl, gl, betal)


_MESH = plsc.VectorSubcoreMesh(core_axis_name="c", subcore_axis_name="s")


@functools.partial(
    pl.kernel,
    mesh=_MESH,
    out_type=jax.ShapeDtypeStruct((_N, _D), jnp.float32),
    scratch_types=[
        pltpu.VMEM((_K, _CH), jnp.int32),
        pltpu.VMEM((_K, _CH), jnp.int32),
        pltpu.VMEM((_K, _CH), jnp.int32),
        pltpu.VMEM((_CB, _D), jnp.float32),
        pltpu.VMEM((_CB, _D), jnp.float32),
        pltpu.VMEM((_ZPT, _D), jnp.float32),
        pltpu.VMEM_SHARED((_AGR, _D), jnp.float32),
    ],
)
def _sc_aggr(x_hbm, e_hbm, src_hbm, dst_hbm, eidx_hbm, out_hbm,
             sidx, didx, eidx, ebuf, gbuf, zbuf, aggr_sh):
    c = lax.axis_index("c")
    s = lax.axis_index("s")

    zero = jnp.zeros((16,), jnp.float32)

    def zfill(i, _):
        for kk in range(_D // 16):
            zbuf[i, pl.ds(kk * 16, 16)] = zero
        return 0

    lax.fori_loop(0, _ZPT, zfill, 0)

    base_cnt = _NCHUNK // _TILES
    rem = _NCHUNK % _TILES
    start = s * base_cnt + jnp.minimum(s, rem)
    cnt = base_cnt + (s < rem).astype(jnp.int32)

    qoff = 0
    for p in range(2):
        qsz = _Q[p]
        dst_off = c * _HALF + qoff

        pltpu.sync_copy(zbuf, aggr_sh.at[pl.ds(s * _ZPT, _ZPT)])
        plsc.subcore_barrier()

        def body(i, _):
            ch = start + i
            pltpu.sync_copy(src_hbm.at[ch], sidx)
            pltpu.sync_copy(dst_hbm.at[ch], didx)
            pltpu.sync_copy(eidx_hbm.at[ch], eidx)
            for j in range(_K):
                pltpu.sync_copy(
                    e_hbm.at[eidx.at[j]], ebuf.at[pl.ds(j * _CH, _CH)]
                )
                pltpu.sync_copy(
                    x_hbm.at[sidx.at[j]], gbuf.at[pl.ds(j * _CH, _CH)]
                )
            for j in range(_K):
                for q in range(_CH // 16):
                    sl = pl.ds(q * 16, 16)
                    d = didx[j, sl] - dst_off
                    ok = (d >= 0) & (d < qsz)
                    didx[j, sl] = jnp.where(ok, d, _TRASH)

            def relu_row(r, _):
                for kk in range(_D // 16):
                    sl = pl.ds(kk * 16, 16)
                    gbuf[r, sl] = jnp.maximum(gbuf[r, sl] + ebuf[r, sl], 0.0)
                return 0

            lax.fori_loop(0, _CB, relu_row, 0)
            for j in range(_K):
                pltpu.sync_copy(
                    gbuf.at[pl.ds(j * _CH, _CH)],
                    aggr_sh.at[didx.at[j]],
                    add=True,
                )
            return 0

        lax.fori_loop(0, cnt, body, 0)
        plsc.subcore_barrier()
        pltpu.sync_copy(
            aggr_sh.at[pl.ds(s * _RPT, _RPT)],
            out_hbm.at[pl.ds(dst_off + s * _RPT, _RPT)],
        )

        @pl.when(s == _TILES - 1)
        def _():
            pltpu.sync_copy(
                aggr_sh.at[pl.ds(_TILES * _RPT, qsz - _TILES * _RPT)],
                out_hbm.at[pl.ds(dst_off + _TILES * _RPT, qsz - _TILES * _RPT)],
            )

        plsc.subcore_barrier()
        qoff += qsz


def kernel(x, edge_index, edge_attr, batch, We, be, W, b, gamma, beta):
    src4 = edge_index[0].reshape(_NCHUNK, _K, _CH)
    dst4 = edge_index[1].reshape(_NCHUNK, _K, _CH)
    eidx0 = jnp.arange(_E, dtype=jnp.int32).reshape(_NCHUNK, _K, _CH)

    e_cat = _edge_enc(edge_attr, We, be.reshape(_L, 1, _D))

    b3 = b.reshape(_L, 1, _D)
    g3 = gamma.reshape(_L, 1, _D)
    bb3 = beta.reshape(_L, 1, _D)

    n_layers = jnp.int32(_L) + (batch[0] >> 31)

    def cond(st):
        return st[0] < n_layers

    def body(st):
        i, prev, xf, eidx_l = st
        Wl = lax.dynamic_slice(W, (i, 0, 0), (1, _D, _D))[0]
        bl = lax.dynamic_slice(b3, (i, 0, 0), (1, 1, _D))[0]
        gl = lax.dynamic_slice(g3, (i, 0, 0), (1, 1, _D))[0]
        bbl = lax.dynamic_slice(bb3, (i, 0, 0), (1, 1, _D))[0]
        aggr = _sc_aggr(xf, e_cat, src4, dst4, eidx_l)
        prev2, xf2 = _update(xf, aggr, prev, Wl, bl, gl, bbl)
        return (i + jnp.int32(1), prev2, xf2, eidx_l + jnp.int32(_E))

    _, _, xf, _ = lax.while_loop(cond, body, (jnp.int32(0), x, x, eidx0))
    return xf


# speedup vs baseline: 1.1023x; 1.1023x over previous
import functools

import jax
import jax.numpy as jnp
from jax import lax
from jax.experimental import pallas as pl
from jax.experimental.pallas import tpu as pltpu
from jax.experimental.pallas import tpu_sc as plsc

_N = 10000
_E = 320000
_D = 128
_DE = 16
_L = 3
_BN_EPS = 1e-5

_CH = 128
_NCHUNK = _E // _CH
_TILES = 16
_PB = _NCHUNK // 2 // _TILES
_HALF = _N // 2
_Q = (2504, 2496)
_AGR = 2512
_TRASH = 2504
_ZPT = _AGR // _TILES
_RPT = 152

_BE = 8000


def _edge_enc(edge_attr, We, be3):
    def body(ea_ref, w_ref, b_ref, o_ref):
        o_ref[...] = (
            jnp.dot(ea_ref[...], w_ref[0], preferred_element_type=jnp.float32)
            + b_ref[0]
        )

    nblk = _E // _BE
    return pl.pallas_call(
        body,
        grid=(_L, nblk),
        in_specs=[
            pl.BlockSpec((_BE, _DE), lambda l, i: (i, 0)),
            pl.BlockSpec((1, _DE, _D), lambda l, i: (l, 0, 0)),
            pl.BlockSpec((1, 1, _D), lambda l, i: (l, 0, 0)),
        ],
        out_specs=pl.BlockSpec((_BE, _D), lambda l, i: (l * nblk + i, 0)),
        out_shape=jax.ShapeDtypeStruct((_L * _E, _D), jnp.float32),
    )(edge_attr, We, be3)


def _update(x, aggr, prev, Wl, bl, gl, betal):
    def body(x_ref, a_ref, p_ref, w_ref, b_ref, g_ref, bb_ref,
             hp_ref, xf_ref):
        h = (
            jnp.dot(x_ref[...] + a_ref[...], w_ref[...],
                    preferred_element_type=jnp.float32)
            + b_ref[...]
        )
        m = jnp.mean(h, axis=0, keepdims=True)
        v = jnp.mean((h - m) ** 2, axis=0, keepdims=True)
        hn = (h - m) * lax.rsqrt(v + _BN_EPS) * g_ref[...] + bb_ref[...]
        hp = hn + p_ref[...]
        hp_ref[...] = hp
        xf_ref[...] = jnp.maximum(hp, 0.0)

    return pl.pallas_call(
        body,
        out_shape=(
            jax.ShapeDtypeStruct((_N, _D), jnp.float32),
            jax.ShapeDtypeStruct((_N, _D), jnp.float32),
        ),
    )(x, aggr, prev, Wl, bl, gl, betal)


_MESH = plsc.VectorSubcoreMesh(core_axis_name="c", subcore_axis_name="s")


@functools.partial(
    pl.kernel,
    mesh=_MESH,
    out_type=jax.ShapeDtypeStruct((_N, _D), jnp.float32),
    scratch_types=[
        pltpu.VMEM((1, _CH), jnp.int32),
        pltpu.VMEM((1, _CH), jnp.int32),
        pltpu.VMEM((1, _CH), jnp.int32),
        pltpu.VMEM((1, _CH), jnp.int32),
        pltpu.VMEM((_CH, _D), jnp.float32),
        pltpu.VMEM((_CH, _D), jnp.float32),
        pltpu.VMEM((_CH, _D), jnp.float32),
        pltpu.VMEM((_ZPT, _D), jnp.float32),
        pltpu.VMEM_SHARED((_AGR, _D), jnp.float32),
        pltpu.SemaphoreType.DMA,
        pltpu.SemaphoreType.DMA,
        pltpu.SemaphoreType.DMA,
    ],
)
def _sc_aggr(x_hbm, e_hbm, src_hbm, dst_hbm, eidx_hbm, out_hbm,
             sidx, didx, eix, scx, ebuf, gbuf, mbuf, zbuf, aggr_sh,
             sem_e, sem_g, sem_s):
    c = lax.axis_index("c")
    s = lax.axis_index("s")

    zero = jnp.zeros((16,), jnp.float32)

    def zfill(i, _):
        for kk in range(_D // 16):
            zbuf[i, pl.ds(kk * 16, 16)] = zero
        return 0

    lax.fori_loop(0, _ZPT, zfill, 0)

    base_cnt = _NCHUNK // _TILES
    rem = _NCHUNK % _TILES
    start = s * base_cnt + jnp.minimum(s, rem)
    cnt = base_cnt + (s < rem).astype(jnp.int32)

    qoff = 0
    for p in range(2):
        qsz = _Q[p]
        dst_off = c * _HALF + qoff

        pltpu.sync_copy(zbuf, aggr_sh.at[pl.ds(s * _ZPT, _ZPT)])
        plsc.subcore_barrier()

        def body(i, _):
            ch = start + i
            pltpu.sync_copy(src_hbm.at[ch], sidx)
            pltpu.sync_copy(dst_hbm.at[ch], didx)
            pltpu.sync_copy(eidx_hbm.at[ch], eix)
            pltpu.async_copy(e_hbm.at[eix.at[0]], ebuf, sem_e)
            pltpu.async_copy(x_hbm.at[sidx.at[0]], gbuf, sem_g)

            @pl.when(i > 0)
            def _():
                pltpu.make_async_copy(
                    mbuf, aggr_sh.at[scx.at[0]], sem_s
                ).wait()

            for q in range(_CH // 16):
                sl = pl.ds(q * 16, 16)
                d = didx[0, sl] - dst_off
                ok = (d >= 0) & (d < qsz)
                scx[0, sl] = jnp.where(ok, d, _TRASH)

            pltpu.make_async_copy(e_hbm.at[eix.at[0]], ebuf, sem_e).wait()
            pltpu.make_async_copy(x_hbm.at[sidx.at[0]], gbuf, sem_g).wait()

            def relu_row(r, _):
                for kk in range(_D // 16):
                    sl = pl.ds(kk * 16, 16)
                    mbuf[r, sl] = jnp.maximum(gbuf[r, sl] + ebuf[r, sl], 0.0)
                return 0

            lax.fori_loop(0, _CH, relu_row, 0)
            pltpu.async_copy(mbuf, aggr_sh.at[scx.at[0]], sem_s, add=True)
            return 0

        lax.fori_loop(0, cnt, body, 0)
        pltpu.make_async_copy(mbuf, aggr_sh.at[scx.at[0]], sem_s).wait()
        plsc.subcore_barrier()
        pltpu.sync_copy(
            aggr_sh.at[pl.ds(s * _RPT, _RPT)],
            out_hbm.at[pl.ds(dst_off + s * _RPT, _RPT)],
        )

        @pl.when(s == _TILES - 1)
        def _():
            pltpu.sync_copy(
                aggr_sh.at[pl.ds(_TILES * _RPT, qsz - _TILES * _RPT)],
                out_hbm.at[pl.ds(dst_off + _TILES * _RPT, qsz - _TILES * _RPT)],
            )

        plsc.subcore_barrier()
        qoff += qsz


def kernel(x, edge_index, edge_attr, batch, We, be, W, b, gamma, beta):
    src4 = edge_index[0].reshape(_NCHUNK, 1, _CH)
    dst4 = edge_index[1].reshape(_NCHUNK, 1, _CH)
    eidx = (
        jnp.arange(_L, dtype=jnp.int32)[:, None] * _E
        + jnp.arange(_E, dtype=jnp.int32)[None, :]
    ).reshape(_L, _NCHUNK, 1, _CH)

    e_cat = _edge_enc(edge_attr, We, be.reshape(_L, 1, _D))

    b3 = b.reshape(_L, 1, _D)
    g3 = gamma.reshape(_L, 1, _D)
    bb3 = beta.reshape(_L, 1, _D)

    n_layers = jnp.int32(_L) + (batch[0] >> 31)

    def cond(st):
        return st[0] < n_layers

    def body(st):
        i, prev, xf = st
        eidx_l = lax.dynamic_slice(
            eidx, (i, 0, 0, 0), (1, _NCHUNK, 1, _CH)
        )[0]
        Wl = lax.dynamic_slice(W, (i, 0, 0), (1, _D, _D))[0]
        bl = lax.dynamic_slice(b3, (i, 0, 0), (1, 1, _D))[0]
        gl = lax.dynamic_slice(g3, (i, 0, 0), (1, 1, _D))[0]
        bbl = lax.dynamic_slice(bb3, (i, 0, 0), (1, 1, _D))[0]
        aggr = _sc_aggr(xf, e_cat, src4, dst4, eidx_l)
        prev2, xf2 = _update(xf, aggr, prev, Wl, bl, gl, bbl)
        return (i + jnp.int32(1), prev2, xf2)

    _, _, xf = lax.while_loop(cond, body, (jnp.int32(0), x, x))
    return xf


# speedup vs baseline: 3.6343x; 3.2971x over previous
import functools

import jax
import jax.numpy as jnp
from jax import lax
from jax.experimental import pallas as pl
from jax.experimental.pallas import tpu as pltpu
from jax.experimental.pallas import tpu_sc as plsc

_N = 10000
_E = 320000
_D = 128
_DE = 16
_L = 3
_BN_EPS = 1e-5

_CH = 128
_NCHUNK = _E // _CH
_CPC = _NCHUNK // 2
_TILES = 16
_ZR = 25
_ZPT = _N // _TILES
_RPT = 624

_BE = 8000


def _edge_enc(edge_attr, We, be3):
    def body(ea_ref, w_ref, b_ref, o_ref):
        o_ref[...] = (
            jnp.dot(ea_ref[...], w_ref[0], preferred_element_type=jnp.float32)
            + b_ref[0]
        )

    nblk = _E // _BE
    return pl.pallas_call(
        body,
        grid=(_L, nblk),
        in_specs=[
            pl.BlockSpec((_BE, _DE), lambda l, i: (i, 0)),
            pl.BlockSpec((1, _DE, _D), lambda l, i: (l, 0, 0)),
            pl.BlockSpec((1, 1, _D), lambda l, i: (l, 0, 0)),
        ],
        out_specs=pl.BlockSpec((_BE, _D), lambda l, i: (l * nblk + i, 0)),
        out_shape=jax.ShapeDtypeStruct((_L * _E, _D), jnp.float32),
    )(edge_attr, We, be3)


def _update(x, aggr, prev, Wl, bl, gl, betal):
    def body(x_ref, a_ref, p_ref, w_ref, b_ref, g_ref, bb_ref,
             hp_ref, xf_ref):
        ag = a_ref[0] + a_ref[1]
        h = (
            jnp.dot(x_ref[...] + ag, w_ref[...],
                    preferred_element_type=jnp.float32)
            + b_ref[...]
        )
        m = jnp.mean(h, axis=0, keepdims=True)
        v = jnp.mean((h - m) ** 2, axis=0, keepdims=True)
        hn = (h - m) * lax.rsqrt(v + _BN_EPS) * g_ref[...] + bb_ref[...]
        hp = hn + p_ref[...]
        hp_ref[...] = hp
        xf_ref[...] = jnp.maximum(hp, 0.0)

    return pl.pallas_call(
        body,
        out_shape=(
            jax.ShapeDtypeStruct((_N, _D), jnp.float32),
            jax.ShapeDtypeStruct((_N, _D), jnp.float32),
        ),
    )(x, aggr, prev, Wl, bl, gl, betal)


_MESH = plsc.VectorSubcoreMesh(core_axis_name="c", subcore_axis_name="s")


def _make_sc_aggr(l):
    @functools.partial(
        pl.kernel,
        mesh=_MESH,
        out_type=jax.ShapeDtypeStruct((2, _N, _D), jnp.float32),
        scratch_types=[
            pltpu.VMEM((2, 1, _CH), jnp.int32),
            pltpu.VMEM((_CH, _D), jnp.float32),
            pltpu.VMEM((_CH, _D), jnp.float32),
            pltpu.VMEM((_ZR, _D), jnp.float32),
            pltpu.VMEM_SHARED((_N, _D), jnp.float32),
            pltpu.SemaphoreType.DMA,
            pltpu.SemaphoreType.DMA,
        ],
    )
    def k(x_hbm, e_hbm, sd_hbm, out_hbm,
          sd, ebuf, gbuf, zbuf, aggr_sh, sem_e, sem_g):
        c = lax.axis_index("c")
        s = lax.axis_index("s")

        zero = jnp.zeros((16,), jnp.float32)

        def zfill(i, _):
            for kk in range(_D // 16):
                zbuf[i, pl.ds(kk * 16, 16)] = zero
            return 0

        lax.fori_loop(0, _ZR, zfill, 0)
        for t in range(_ZPT // _ZR):
            pltpu.sync_copy(zbuf, aggr_sh.at[pl.ds(s * _ZPT + t * _ZR, _ZR)])
        plsc.subcore_barrier()

        base_cnt = _CPC // _TILES
        rem = _CPC % _TILES
        start = c * _CPC + s * base_cnt + jnp.minimum(s, rem)
        cnt = base_cnt + (s < rem).astype(jnp.int32)

        def body(i, _):
            ch = start + i
            pltpu.sync_copy(sd_hbm.at[ch], sd)
            pltpu.async_copy(
                e_hbm.at[pl.ds(l * _E + ch * _CH, _CH)], ebuf, sem_e
            )
            pltpu.async_copy(x_hbm.at[sd.at[0, 0]], gbuf, sem_g)
            pltpu.make_async_copy(
                e_hbm.at[pl.ds(l * _E + ch * _CH, _CH)], ebuf, sem_e
            ).wait()
            pltpu.make_async_copy(
                x_hbm.at[sd.at[0, 0]], gbuf, sem_g
            ).wait()

            def relu_row(r, _):
                for kk in range(_D // 16):
                    sl = pl.ds(kk * 16, 16)
                    gbuf[r, sl] = jnp.maximum(gbuf[r, sl] + ebuf[r, sl], 0.0)
                return 0

            lax.fori_loop(0, _CH, relu_row, 0)
            pltpu.sync_copy(gbuf, aggr_sh.at[sd.at[1, 0]], add=True)
            return 0

        lax.fori_loop(0, cnt, body, 0)
        plsc.subcore_barrier()
        pltpu.sync_copy(
            aggr_sh.at[pl.ds(s * _RPT, _RPT)],
            out_hbm.at[c, pl.ds(s * _RPT, _RPT)],
        )

        @pl.when(s == _TILES - 1)
        def _():
            pltpu.sync_copy(
                aggr_sh.at[pl.ds(_TILES * _RPT, _N - _TILES * _RPT)],
                out_hbm.at[c, pl.ds(_TILES * _RPT, _N - _TILES * _RPT)],
            )

    return k


_SC_AGGR = [_make_sc_aggr(l) for l in range(_L)]


def kernel(x, edge_index, edge_attr, batch, We, be, W, b, gamma, beta):
    del batch
    sd4 = jnp.stack(
        [
            edge_index[0].reshape(_NCHUNK, 1, _CH),
            edge_index[1].reshape(_NCHUNK, 1, _CH),
        ],
        axis=1,
    )

    e_cat = _edge_enc(edge_attr, We, be.reshape(_L, 1, _D))

    prev = x
    xf = x
    for l in range(_L):
        aggr = _SC_AGGR[l](xf, e_cat, sd4)
        prev, xf = _update(
            xf, aggr, prev, W[l],
            b[l].reshape(1, _D), gamma[l].reshape(1, _D), beta[l].reshape(1, _D),
        )
    return xf


# speedup vs baseline: 4.0251x; 1.1075x over previous
import functools

import jax
import jax.numpy as jnp
from jax import lax
from jax.experimental import pallas as pl
from jax.experimental.pallas import tpu as pltpu
from jax.experimental.pallas import tpu_sc as plsc

_N = 10000
_E = 320000
_D = 128
_DE = 16
_L = 3
_BN_EPS = 1e-5

_CH = 128
_NCHUNK = _E // _CH
_CPC = _NCHUNK // 2
_TILES = 16
_ZR = 25
_ZPT = _N // _TILES
_RPT = 624

_BE = 8000


def _edge_enc(edge_attr, We, be3):
    def body(ea_ref, w_ref, b_ref, o_ref):
        o_ref[...] = (
            jnp.dot(ea_ref[...], w_ref[0], preferred_element_type=jnp.float32)
            + b_ref[0]
        )

    nblk = _E // _BE
    return pl.pallas_call(
        body,
        grid=(_L, nblk),
        in_specs=[
            pl.BlockSpec((_BE, _DE), lambda l, i: (i, 0)),
            pl.BlockSpec((1, _DE, _D), lambda l, i: (l, 0, 0)),
            pl.BlockSpec((1, 1, _D), lambda l, i: (l, 0, 0)),
        ],
        out_specs=pl.BlockSpec((_BE, _D), lambda l, i: (l * nblk + i, 0)),
        out_shape=jax.ShapeDtypeStruct((_L * _E, _D), jnp.float32),
    )(edge_attr, We, be3)


def _update(x, aggr, prev, Wl, bl, gl, betal):
    def body(x_ref, a_ref, p_ref, w_ref, b_ref, g_ref, bb_ref,
             hp_ref, xf_ref):
        ag = a_ref[0] + a_ref[1]
        h = (
            jnp.dot(x_ref[...] + ag, w_ref[...],
                    preferred_element_type=jnp.float32)
            + b_ref[...]
        )
        m = jnp.mean(h, axis=0, keepdims=True)
        v = jnp.mean((h - m) ** 2, axis=0, keepdims=True)
        hn = (h - m) * lax.rsqrt(v + _BN_EPS) * g_ref[...] + bb_ref[...]
        hp = hn + p_ref[...]
        hp_ref[...] = hp
        xf_ref[...] = jnp.maximum(hp, 0.0)

    return pl.pallas_call(
        body,
        out_shape=(
            jax.ShapeDtypeStruct((_N, _D), jnp.float32),
            jax.ShapeDtypeStruct((_N, _D), jnp.float32),
        ),
    )(x, aggr, prev, Wl, bl, gl, betal)


_MESH = plsc.VectorSubcoreMesh(core_axis_name="c", subcore_axis_name="s")


def _make_sc_aggr(l):
    @functools.partial(
        pl.kernel,
        mesh=_MESH,
        out_type=jax.ShapeDtypeStruct((2, _N, _D), jnp.float32),
        scratch_types=[
            pltpu.VMEM((2, 1, _CH), jnp.int32),
            pltpu.VMEM((1, _CH), jnp.int32),
            pltpu.VMEM((_CH, _D), jnp.float32),
            pltpu.VMEM((_CH, _D), jnp.float32),
            pltpu.VMEM((_CH, _D), jnp.float32),
            pltpu.VMEM_SHARED((_N, _D), jnp.float32),
            pltpu.SemaphoreType.DMA,
            pltpu.SemaphoreType.DMA,
            pltpu.SemaphoreType.DMA,
        ],
    )
    def k(x_hbm, e_hbm, sd_hbm, out_hbm,
          sd, scx, ebuf, gbuf, mbuf, aggr_sh, sem_e, sem_g, sem_s):
        c = lax.axis_index("c")
        s = lax.axis_index("s")

        zero = jnp.zeros((16,), jnp.float32)

        def zfill(i, _):
            for kk in range(_D // 16):
                mbuf[i, pl.ds(kk * 16, 16)] = zero
            return 0

        lax.fori_loop(0, _CH, zfill, 0)
        for t in range(_ZPT // _CH):
            pltpu.sync_copy(
                mbuf, aggr_sh.at[pl.ds(s * _ZPT + t * _CH, _CH)]
            )
        pltpu.sync_copy(
            mbuf.at[pl.ds(0, _ZPT % _CH)],
            aggr_sh.at[pl.ds(s * _ZPT + (_ZPT // _CH) * _CH, _ZPT % _CH)],
        )
        plsc.subcore_barrier()

        base_cnt = _CPC // _TILES
        rem = _CPC % _TILES
        start = c * _CPC + s * base_cnt + jnp.minimum(s, rem)
        cnt = base_cnt + (s < rem).astype(jnp.int32)

        def body(i, _):
            ch = start + i
            pltpu.sync_copy(sd_hbm.at[ch], sd)
            pltpu.async_copy(
                e_hbm.at[pl.ds(l * _E + ch * _CH, _CH)], ebuf, sem_e
            )
            pltpu.async_copy(x_hbm.at[sd.at[0, 0]], gbuf, sem_g)

            @pl.when(i > 0)
            def _():
                pltpu.make_async_copy(
                    mbuf, aggr_sh.at[scx.at[0]], sem_s
                ).wait()

            for kk in range(_CH // 16):
                sl = pl.ds(kk * 16, 16)
                scx[0, sl] = sd[1, 0, sl]

            pltpu.make_async_copy(
                e_hbm.at[pl.ds(l * _E + ch * _CH, _CH)], ebuf, sem_e
            ).wait()
            pltpu.make_async_copy(
                x_hbm.at[sd.at[0, 0]], gbuf, sem_g
            ).wait()

            def relu_row(r, _):
                for u in range(2):
                    for kk in range(_D // 16):
                        sl = pl.ds(kk * 16, 16)
                        mbuf[2 * r + u, sl] = jnp.maximum(
                            gbuf[2 * r + u, sl] + ebuf[2 * r + u, sl], 0.0
                        )
                return 0

            lax.fori_loop(0, _CH // 2, relu_row, 0)
            pltpu.async_copy(mbuf, aggr_sh.at[scx.at[0]], sem_s, add=True)
            return 0

        lax.fori_loop(0, cnt, body, 0)
        pltpu.make_async_copy(mbuf, aggr_sh.at[scx.at[0]], sem_s).wait()
        plsc.subcore_barrier()
        pltpu.sync_copy(
            aggr_sh.at[pl.ds(s * _RPT, _RPT)],
            out_hbm.at[c, pl.ds(s * _RPT, _RPT)],
        )

        @pl.when(s == _TILES - 1)
        def _():
            pltpu.sync_copy(
                aggr_sh.at[pl.ds(_TILES * _RPT, _N - _TILES * _RPT)],
                out_hbm.at[c, pl.ds(_TILES * _RPT, _N - _TILES * _RPT)],
            )

    return k


_SC_AGGR = [_make_sc_aggr(l) for l in range(_L)]


def kernel(x, edge_index, edge_attr, batch, We, be, W, b, gamma, beta):
    del batch
    sd4 = jnp.stack(
        [
            edge_index[0].reshape(_NCHUNK, 1, _CH),
            edge_index[1].reshape(_NCHUNK, 1, _CH),
        ],
        axis=1,
    )

    e_cat = _edge_enc(edge_attr, We, be.reshape(_L, 1, _D))

    prev = x
    xf = x
    for l in range(_L):
        aggr = _SC_AGGR[l](xf, e_cat, sd4)
        prev, xf = _update(
            xf, aggr, prev, W[l],
            b[l].reshape(1, _D), gamma[l].reshape(1, _D), beta[l].reshape(1, _D),
        )
    return xf
